# baseline (speedup 1.0000x reference)
import functools

import jax
import jax.numpy as jnp
from jax import lax
from jax.experimental import pallas as pl
from jax.experimental.pallas import tpu as pltpu
from jax.experimental.pallas import tpu_sc as plsc

VOCAB = 100000
HIDDEN = 64
NUM_PIXEL = 4096
BATCH = 64

NUM_CORES = 2
NUM_SUBCORES = 16
NUM_WORKERS = NUM_CORES * NUM_SUBCORES
PPW = NUM_PIXEL // NUM_WORKERS
LANES = 16
NBUF = 2

_mesh = plsc.VectorSubcoreMesh(core_axis_name="c", subcore_axis_name="s")


@functools.partial(
    pl.kernel,
    out_type=jax.ShapeDtypeStruct((BATCH, NUM_PIXEL, HIDDEN), jnp.float32),
    mesh=_mesh,
    scratch_types=[
        pltpu.VMEM((BATCH, PPW), jnp.int32),
        pltpu.VMEM((PPW // 2, 2 * HIDDEN), jnp.float32),
        pltpu.VMEM((NBUF, PPW), jnp.int32),
        pltpu.VMEM((NBUF, PPW), jnp.int32),
        pltpu.VMEM((NBUF, PPW, 2 * HIDDEN), jnp.float32),
        pltpu.VMEM((NBUF, PPW, HIDDEN), jnp.float32),
        [pltpu.SemaphoreType.DMA] * NBUF,
        [pltpu.SemaphoreType.DMA] * NBUF,
    ],
)
def _emb_kernel(idx_hbm, table_hbm, pos_hbm, out_hbm,
                idx_v, pos_v, idxh_v, off_v, gbuf_v, obuf_v, gsems, osems):
    c = lax.axis_index("c")
    s = lax.axis_index("s")
    w = s * NUM_CORES + c
    base = w * PPW

    pltpu.sync_copy(idx_hbm.at[w], idx_v)
    pltpu.sync_copy(pos_hbm.at[pl.ds(w * (PPW // 2), PPW // 2), :], pos_v)

    def gather(d):
        return pltpu.make_async_copy(
            table_hbm.at[idxh_v.at[d]], gbuf_v.at[d], gsems[d])

    def out_copy(b, d):
        return pltpu.make_async_copy(
            obuf_v.at[d], out_hbm.at[b, pl.ds(base, PPW), :], osems[d])

    def prep(b, d):
        @pl.loop(0, PPW // LANES)
        def _split(j):
            v = idx_v[b, pl.ds(j * LANES, LANES)]
            idxh_v[d, pl.ds(j * LANES, LANES)] = lax.shift_right_logical(v, 1)
            off_v[d, pl.ds(j * LANES, LANES)] = lax.shift_left(
                lax.bitwise_and(v, 1), 6)
        gather(d).start()

    for d in range(NBUF):
        prep(d, d)

    @pl.loop(0, BATCH, step=NBUF)
    def _group(g):
        for d in range(NBUF):
            b = g + d
            gather(d).wait()

            @pl.when(b >= NBUF)
            def _():
                out_copy(b - NBUF, d).wait()

            @pl.loop(0, PPW // LANES)
            def _rowgrp(g):
                ov = off_v[d, pl.ds(g * LANES, LANES)]
                for j in range(LANES):
                    i = g * LANES + j
                    o = ov[j]
                    q = (j & 1) * HIDDEN
                    ph = g * (LANES // 2) + j // 2
                    for k in range(HIDDEN // LANES):
                        t = gbuf_v[d, i, pl.ds(o + k * LANES, LANES)]
                        p = pos_v[ph, pl.ds(q + k * LANES, LANES)]
                        obuf_v[d, i, pl.ds(k * LANES, LANES)] = t + p

            out_copy(b, d).start()

            @pl.when(b + NBUF < BATCH)
            def _():
                prep(b + NBUF, d)

    for d in range(NBUF):
        out_copy(BATCH - NBUF + d, d).wait()


def kernel(pixel_values, token_table, position_embedding):
    idx = pixel_values.astype(jnp.int32)
    idx_t = jnp.transpose(idx.reshape(BATCH, NUM_WORKERS, PPW), (1, 0, 2))
    table2 = token_table.reshape(VOCAB // 2, 2 * HIDDEN)
    pos2 = position_embedding.reshape(NUM_PIXEL // 2, 2 * HIDDEN)
    return _emb_kernel(idx_t, table2, pos2)


# speedup vs baseline: 1.8749x; 1.8749x over previous
import functools

import jax
import jax.numpy as jnp
from jax import lax
from jax.experimental import pallas as pl
from jax.experimental.pallas import tpu as pltpu
from jax.experimental.pallas import tpu_sc as plsc

VOCAB = 100000
HIDDEN = 64
NUM_PIXEL = 4096
BATCH = 64

NUM_CORES = 2
NUM_SUBCORES = 16
NUM_WORKERS = NUM_CORES * NUM_SUBCORES
PPW = NUM_PIXEL // NUM_WORKERS
LANES = 16
NBUF = 4

_mesh = plsc.VectorSubcoreMesh(core_axis_name="c", subcore_axis_name="s")


@functools.partial(
    pl.kernel,
    out_type=jax.ShapeDtypeStruct((BATCH, NUM_PIXEL, 2 * HIDDEN), jnp.float32),
    mesh=_mesh,
    scratch_types=[
        pltpu.VMEM((BATCH, PPW), jnp.int32),
        pltpu.VMEM((PPW // 2, 2 * HIDDEN), jnp.float32),
        pltpu.VMEM((NBUF, PPW, HIDDEN), jnp.float32),
        pltpu.VMEM((NBUF, PPW, HIDDEN), jnp.float32),
        [pltpu.SemaphoreType.DMA] * NBUF,
        [pltpu.SemaphoreType.DMA] * NBUF,
    ],
    compiler_params=pltpu.CompilerParams(use_tc_tiling_on_sc=False),
)
def _emb_kernel(idx_hbm, table_hbm, pos_hbm, out_hbm,
                idx_v, pos_v, rows_v, obuf_v, gsems, osems):
    c = lax.axis_index("c")
    s = lax.axis_index("s")
    w = s * NUM_CORES + c
    base = w * PPW

    pltpu.sync_copy(idx_hbm.at[w], idx_v)
    pltpu.sync_copy(pos_hbm.at[pl.ds(w * (PPW // 2), PPW // 2), :], pos_v)

    def gather(b, d):
        return pltpu.make_async_copy(
            table_hbm.at[idx_v.at[b]], rows_v.at[d], gsems[d])

    def out_copy(b, d):
        return pltpu.make_async_copy(
            obuf_v.at[d],
            out_hbm.at[b, pl.ds(base, PPW), pl.ds(0, HIDDEN)],
            osems[d])

    for d in range(NBUF):
        gather(d, d).start()

    @pl.loop(0, BATCH, step=NBUF)
    def _group(g):
        for d in range(NBUF):
            b = g + d
            gather(b, d).wait()

            @pl.when(b >= NBUF)
            def _():
                out_copy(b - NBUF, d).wait()

            @pl.loop(0, PPW // LANES)
            def _rowgrp(gi):
                for j in range(LANES):
                    i = gi * LANES + j
                    q = (j & 1) * HIDDEN
                    ph = gi * (LANES // 2) + j // 2
                    for k in range(HIDDEN // LANES):
                        t = rows_v[d, i, pl.ds(k * LANES, LANES)]
                        p = pos_v[ph, pl.ds(q + k * LANES, LANES)]
                        obuf_v[d, i, pl.ds(k * LANES, LANES)] = t + p

            out_copy(b, d).start()

            @pl.when(b + NBUF < BATCH)
            def _():
                gather(b + NBUF, d).start()

    for d in range(NBUF):
        out_copy(BATCH - NBUF + d, d).wait()


def kernel(pixel_values, token_table, position_embedding):
    idx = pixel_values.astype(jnp.int32)
    idx_t = jnp.transpose(idx.reshape(BATCH, NUM_WORKERS, PPW), (1, 0, 2))
    pos2 = position_embedding.reshape(NUM_PIXEL // 2, 2 * HIDDEN)
    out = _emb_kernel(idx_t, token_table, pos2)
    return out[:, :, :HIDDEN]
